# baseline (speedup 1.0000x reference)
import functools

import jax
import jax.numpy as jnp
from jax import lax
from jax.experimental import pallas as pl
from jax.experimental.pallas import tpu as pltpu

_CL = 32


def _stats_kernel(feat_ref, lab_ref, sumf_ref, sumsq_ref, cnt_ref):
    s = pl.program_id(1)

    @pl.when(s == 0)
    def _():
        sumf_ref[...] = jnp.zeros_like(sumf_ref)
        sumsq_ref[...] = jnp.zeros_like(sumsq_ref)
        cnt_ref[...] = jnp.zeros_like(cnt_ref)

    f = feat_ref[...]
    lab = lab_ref[...]
    tb = lab.shape[1]
    oh = (lax.broadcasted_iota(jnp.int32, (_CL, tb), 0) == lab)
    oh = oh.astype(jnp.float32)
    sumf_ref[0] += jnp.dot(oh, f, preferred_element_type=jnp.float32)
    sumsq_ref[0] += jnp.dot(oh, f * f, preferred_element_type=jnp.float32)
    cnt_ref[0] += jnp.sum(oh, axis=1, keepdims=True)


def _aug_kernel(sumf_ref, sumsq_ref, cnt_ref, w32_ref, wt_ref, ratio_ref,
                y_ref, lab_ref, out_ref, st_ref, *, num_classes, n_par):
    s = pl.program_id(1)

    @pl.when(s == 0)
    def _():
        sumf = sumf_ref[0]
        sumsq = sumsq_ref[0]
        cnt = cnt_ref[0]
        for p in range(1, n_par):
            sumf = sumf + sumf_ref[p]
            sumsq = sumsq + sumsq_ref[p]
            cnt = cnt + cnt_ref[p]
        n = jnp.maximum(cnt, 1.0)
        ave = sumf / n
        cv = jnp.maximum(sumsq / n - ave * ave, 0.0)
        row = lax.broadcasted_iota(jnp.int32, cv.shape, 0)
        cv = jnp.where(row < num_classes, cv, 0.0)
        w32 = w32_ref[...]
        wt = wt_ref[...]
        t1 = jnp.dot(cv, wt * wt, preferred_element_type=jnp.float32)
        t2 = jnp.dot(cv * w32, wt, preferred_element_type=jnp.float32)
        t3 = jnp.sum(w32 * w32 * cv, axis=1, keepdims=True)
        q = t1 - 2.0 * t2 + t3
        st_ref[...] = (ratio_ref[0, 0] * q.T)[:num_classes]

    lab = lab_ref[0]
    st_half = 0.5 * st_ref[...]
    parts = []
    for g in range(4):
        accg = None
        for c2 in range(g, num_classes, 4):
            m = (lab == c2).astype(jnp.float32)
            stc = st_half[:, c2:c2 + 1].reshape(st_half.shape[0], 1, 1)
            t = stc * m[None, :, :]
            accg = t if accg is None else accg + t
        parts.append(accg)
    out_ref[0] = y_ref[0] + ((parts[0] + parts[1]) + (parts[2] + parts[3]))


def kernel(features, fc_weight_conv, y, target_x, ratio):
    N, A, H, W = features.shape
    C = fc_weight_conv.shape[0]
    HW = H * W
    NHW = N * HW

    _, h_in, w_in = target_x.shape
    hi = jnp.floor(jnp.arange(H) * (h_in / H)).astype(jnp.int32)
    wi = jnp.floor(jnp.arange(W) * (w_in / W)).astype(jnp.int32)
    lab = target_x.astype(jnp.float32)[:, hi, :][:, :, wi].astype(jnp.int32)
    labm = jnp.where(lab == 255, C, lab).reshape(1, NHW)

    feat_flat = jnp.transpose(features, (0, 2, 3, 1)).reshape(NHW, A)

    n_par = 2
    tb = min(4096, NHW // n_par)
    n_inner = NHW // (tb * n_par)

    sumf, sumsq, cnt = pl.pallas_call(
        _stats_kernel,
        out_shape=(
            jax.ShapeDtypeStruct((n_par, _CL, A), jnp.float32),
            jax.ShapeDtypeStruct((n_par, _CL, A), jnp.float32),
            jax.ShapeDtypeStruct((n_par, _CL, 1), jnp.float32),
        ),
        grid=(n_par, n_inner),
        in_specs=[
            pl.BlockSpec((tb, A), lambda p, s: (p * n_inner + s, 0)),
            pl.BlockSpec((1, tb), lambda p, s: (0, p * n_inner + s)),
        ],
        out_specs=(
            pl.BlockSpec((1, _CL, A), lambda p, s: (p, 0, 0)),
            pl.BlockSpec((1, _CL, A), lambda p, s: (p, 0, 0)),
            pl.BlockSpec((1, _CL, 1), lambda p, s: (p, 0, 0)),
        ),
        compiler_params=pltpu.CompilerParams(
            dimension_semantics=("parallel", "arbitrary")),
    )(feat_flat, labm)

    wm = fc_weight_conv.reshape(C, A)
    w32 = jnp.zeros((_CL, A), jnp.float32).at[:C].set(wm)
    wt = jnp.zeros((A, _CL), jnp.float32).at[:, :C].set(wm.T)
    ratio_arr = jnp.asarray(ratio, jnp.float32).reshape(1, 1)

    hb = H
    n_hblk = H // hb
    n_yinner = N * n_hblk // n_par
    aug = pl.pallas_call(
        functools.partial(_aug_kernel, num_classes=C, n_par=n_par),
        out_shape=jax.ShapeDtypeStruct((N, C, H, W), jnp.float32),
        grid=(n_par, n_yinner),
        in_specs=[
            pl.BlockSpec((n_par, _CL, A), lambda p, s: (0, 0, 0)),
            pl.BlockSpec((n_par, _CL, A), lambda p, s: (0, 0, 0)),
            pl.BlockSpec((n_par, _CL, 1), lambda p, s: (0, 0, 0)),
            pl.BlockSpec((_CL, A), lambda p, s: (0, 0)),
            pl.BlockSpec((A, _CL), lambda p, s: (0, 0)),
            pl.BlockSpec((1, 1), lambda p, s: (0, 0)),
            pl.BlockSpec(
                (1, C, hb, W),
                lambda p, s: ((p * n_yinner + s) // n_hblk, 0,
                              (p * n_yinner + s) % n_hblk, 0)),
            pl.BlockSpec(
                (1, hb, W),
                lambda p, s: ((p * n_yinner + s) // n_hblk,
                              (p * n_yinner + s) % n_hblk, 0)),
        ],
        out_specs=pl.BlockSpec(
            (1, C, hb, W),
            lambda p, s: ((p * n_yinner + s) // n_hblk, 0,
                          (p * n_yinner + s) % n_hblk, 0)),
        scratch_shapes=[pltpu.VMEM((C, _CL), jnp.float32)],
        compiler_params=pltpu.CompilerParams(
            dimension_semantics=("parallel", "arbitrary")),
    )(sumf, sumsq, cnt, w32, wt, ratio_arr, y, lab)

    return aug


# speedup vs baseline: 1.0092x; 1.0092x over previous
import functools

import jax
import jax.numpy as jnp
from jax import lax
from jax.experimental import pallas as pl
from jax.experimental.pallas import tpu as pltpu

_CL = 32


def _stats_kernel(feat_ref, lab_ref, sumf_ref, sumsq_ref, cnt_ref):
    s = pl.program_id(1)

    @pl.when(s == 0)
    def _():
        sumf_ref[...] = jnp.zeros_like(sumf_ref)
        sumsq_ref[...] = jnp.zeros_like(sumsq_ref)
        cnt_ref[...] = jnp.zeros_like(cnt_ref)

    f = feat_ref[...]
    lab = lab_ref[...]
    tb = lab.shape[1]
    oh = (lax.broadcasted_iota(jnp.int32, (_CL, tb), 0) == lab)
    oh = oh.astype(jnp.float32)
    sumf_ref[0] += jnp.dot(oh, f, preferred_element_type=jnp.float32)
    sumsq_ref[0] += jnp.dot(oh, f * f, preferred_element_type=jnp.float32)
    cnt_ref[0] += jnp.sum(oh, axis=1, keepdims=True)


def _aug_kernel(sumf_ref, sumsq_ref, cnt_ref, w32_ref, wt_ref, ratio_ref,
                y_ref, lab_ref, out_ref, st_ref, *, num_classes, n_par):
    s = pl.program_id(1)

    @pl.when(s == 0)
    def _():
        sumf = sumf_ref[0]
        sumsq = sumsq_ref[0]
        cnt = cnt_ref[0]
        for p in range(1, n_par):
            sumf = sumf + sumf_ref[p]
            sumsq = sumsq + sumsq_ref[p]
            cnt = cnt + cnt_ref[p]
        n = jnp.maximum(cnt, 1.0)
        ave = sumf / n
        cv = jnp.maximum(sumsq / n - ave * ave, 0.0)
        row = lax.broadcasted_iota(jnp.int32, cv.shape, 0)
        cv = jnp.where(row < num_classes, cv, 0.0)
        w32 = w32_ref[...]
        wt = wt_ref[...]
        t1 = jnp.dot(cv, wt * wt, preferred_element_type=jnp.float32)
        t2 = jnp.dot(cv * w32, wt, preferred_element_type=jnp.float32)
        t3 = jnp.sum(w32 * w32 * cv, axis=1, keepdims=True)
        q = t1 - 2.0 * t2 + t3
        st_ref[...] = (ratio_ref[0, 0] * q.T)[:num_classes]

    lab = lab_ref[0]
    acc = y_ref[0]
    st_half = 0.5 * st_ref[...]
    for c2 in range(num_classes):
        m = (lab == c2).astype(jnp.float32)
        stc = st_half[:, c2:c2 + 1].reshape(st_half.shape[0], 1, 1)
        acc = acc + stc * m[None, :, :]
    out_ref[0] = acc


def kernel(features, fc_weight_conv, y, target_x, ratio):
    N, A, H, W = features.shape
    C = fc_weight_conv.shape[0]
    HW = H * W
    NHW = N * HW

    _, h_in, w_in = target_x.shape
    hi = jnp.floor(jnp.arange(H) * (h_in / H)).astype(jnp.int32)
    wi = jnp.floor(jnp.arange(W) * (w_in / W)).astype(jnp.int32)
    lab = target_x.astype(jnp.float32)[:, hi, :][:, :, wi].astype(jnp.int32)
    labm = jnp.where(lab == 255, C, lab).reshape(1, NHW)

    feat_flat = jnp.transpose(features, (0, 2, 3, 1)).reshape(NHW, A)

    n_par = 2
    tb = min(4096, NHW // n_par)
    n_inner = NHW // (tb * n_par)

    sumf, sumsq, cnt = pl.pallas_call(
        _stats_kernel,
        out_shape=(
            jax.ShapeDtypeStruct((n_par, _CL, A), jnp.float32),
            jax.ShapeDtypeStruct((n_par, _CL, A), jnp.float32),
            jax.ShapeDtypeStruct((n_par, _CL, 1), jnp.float32),
        ),
        grid=(n_par, n_inner),
        in_specs=[
            pl.BlockSpec((tb, A), lambda p, s: (p * n_inner + s, 0)),
            pl.BlockSpec((1, tb), lambda p, s: (0, p * n_inner + s)),
        ],
        out_specs=(
            pl.BlockSpec((1, _CL, A), lambda p, s: (p, 0, 0)),
            pl.BlockSpec((1, _CL, A), lambda p, s: (p, 0, 0)),
            pl.BlockSpec((1, _CL, 1), lambda p, s: (p, 0, 0)),
        ),
        compiler_params=pltpu.CompilerParams(
            dimension_semantics=("parallel", "arbitrary")),
    )(feat_flat, labm)

    wm = fc_weight_conv.reshape(C, A)
    w32 = jnp.zeros((_CL, A), jnp.float32).at[:C].set(wm)
    wt = jnp.zeros((A, _CL), jnp.float32).at[:, :C].set(wm.T)
    ratio_arr = jnp.asarray(ratio, jnp.float32).reshape(1, 1)

    hb = H
    n_hblk = H // hb
    n_yinner = N * n_hblk // n_par
    aug = pl.pallas_call(
        functools.partial(_aug_kernel, num_classes=C, n_par=n_par),
        out_shape=jax.ShapeDtypeStruct((N, C, H, W), jnp.float32),
        grid=(n_par, n_yinner),
        in_specs=[
            pl.BlockSpec((n_par, _CL, A), lambda p, s: (0, 0, 0)),
            pl.BlockSpec((n_par, _CL, A), lambda p, s: (0, 0, 0)),
            pl.BlockSpec((n_par, _CL, 1), lambda p, s: (0, 0, 0)),
            pl.BlockSpec((_CL, A), lambda p, s: (0, 0)),
            pl.BlockSpec((A, _CL), lambda p, s: (0, 0)),
            pl.BlockSpec((1, 1), lambda p, s: (0, 0)),
            pl.BlockSpec(
                (1, C, hb, W),
                lambda p, s: ((p * n_yinner + s) // n_hblk, 0,
                              (p * n_yinner + s) % n_hblk, 0)),
            pl.BlockSpec(
                (1, hb, W),
                lambda p, s: ((p * n_yinner + s) // n_hblk,
                              (p * n_yinner + s) % n_hblk, 0)),
        ],
        out_specs=pl.BlockSpec(
            (1, C, hb, W),
            lambda p, s: ((p * n_yinner + s) // n_hblk, 0,
                          (p * n_yinner + s) % n_hblk, 0)),
        scratch_shapes=[pltpu.VMEM((C, _CL), jnp.float32)],
        compiler_params=pltpu.CompilerParams(
            dimension_semantics=("parallel", "arbitrary")),
    )(sumf, sumsq, cnt, w32, wt, ratio_arr, y, lab)

    return aug
